# baseline (speedup 1.0000x reference)
import functools

import jax
import jax.numpy as jnp
from jax import lax
from jax.experimental import pallas as pl
from jax.experimental.pallas import tpu as pltpu
from jax.experimental.pallas import tpu_sc as plsc

_NC = 2
_NS = 16
_NW = _NC * _NS
_L = 16


def _lr_body(idx_hbm, xn_hbm, wb_hbm, tbl_hbm, out_hbm,
             idx_v, g_v, xn_v, wb_v, out_v, sem,
             *, bpw, num_fields, num_dim):
  wid = lax.axis_index("s") * _NC + lax.axis_index("c")

  pltpu.sync_copy(idx_hbm.at[wid], idx_v)
  gather = pltpu.async_copy(tbl_hbm.at[idx_v], g_v, sem)
  pltpu.sync_copy(xn_hbm.at[wid], xn_v)
  pltpu.sync_copy(wb_hbm, wb_v)
  gather.wait()

  bias_vec = wb_v[num_dim]

  def chunk_body(j, _):
    base = j * _L

    def f_body(f, acc):
      return acc + g_v[pl.ds(f * bpw + base, _L)]

    acc = lax.fori_loop(0, num_fields, f_body, bias_vec)

    def d_body(d, acc):
      return acc + xn_v[pl.ds(d * bpw + base, _L)] * wb_v[d]

    acc = lax.fori_loop(0, num_dim, d_body, acc)
    out_v[pl.ds(base, _L)] = acc
    return 0

  lax.fori_loop(0, bpw // _L, chunk_body, 0)
  pltpu.sync_copy(out_v, out_hbm.at[pl.ds(wid * bpw, bpw)])


@functools.partial(jax.jit, static_argnames=())
def kernel(x_cat, x_num, tables, W, bias):
  B, F = x_cat.shape
  _, D_NUM = x_num.shape
  V = tables.shape[1]
  bpw = B // _NW

  idx = (x_cat + (jnp.arange(F, dtype=jnp.int32) * V)[None, :])
  idx = idx.T.reshape(F, _NW, bpw).transpose(1, 0, 2).reshape(_NW, F * bpw)
  xn = x_num.T.reshape(D_NUM, _NW, bpw).transpose(1, 0, 2).reshape(
      _NW, D_NUM * bpw)
  tbl = tables.reshape(F * V)
  wb = jnp.concatenate(
      [jnp.broadcast_to(W.reshape(D_NUM, 1), (D_NUM, _L)),
       jnp.broadcast_to(bias.reshape(1, 1), (1, _L))], axis=0)

  mesh = plsc.VectorSubcoreMesh(core_axis_name="c", subcore_axis_name="s",
                                num_cores=_NC, num_subcores=_NS)
  body = functools.partial(_lr_body, bpw=bpw, num_fields=F, num_dim=D_NUM)
  out = pl.kernel(
      body,
      out_type=jax.ShapeDtypeStruct((B,), jnp.float32),
      mesh=mesh,
      scratch_types=[
          pltpu.VMEM((bpw * F,), jnp.int32),
          pltpu.VMEM((bpw * F,), jnp.float32),
          pltpu.VMEM((bpw * D_NUM,), jnp.float32),
          pltpu.VMEM((D_NUM + 1, _L), jnp.float32),
          pltpu.VMEM((bpw,), jnp.float32),
          pltpu.SemaphoreType.DMA,
      ],
  )(idx, xn, wb, tbl)
  return out.reshape(B, 1)
